# baseline (speedup 1.0000x reference)
import jax
import jax.numpy as jnp
from jax import lax
from jax.experimental import pallas as pl

N = 8192
K = 8
BLOCK = 1024
PAD_D = 8
LANES = 128
NCOL = N // LANES


def _lazy(f):
    memo = []

    def g():
        if not memo:
            memo.append(f())
        return memo[0]
    return g


def _leaf(v, i):
    return (lambda: v), (lambda: i)


def _cx(a, b):
    pred = _lazy(lambda: a[0]() <= b[0]())
    lo = (_lazy(lambda: jnp.minimum(a[0](), b[0]())),
          _lazy(lambda: jnp.where(pred(), a[1](), b[1]())))
    hi = (_lazy(lambda: jnp.maximum(a[0](), b[0]())),
          _lazy(lambda: jnp.where(pred(), b[1](), a[1]())))
    return lo, hi


def _oe_merge(a, b):
    if not a:
        return b
    if not b:
        return a
    if len(a) == 1 and len(b) == 1:
        lo, hi = _cx(a[0], b[0])
        return [lo, hi]
    e = _oe_merge(a[::2], b[::2])
    o = _oe_merge(a[1::2], b[1::2])
    res = [e[0]]
    for i in range(len(o)):
        if i + 1 < len(e):
            lo, hi = _cx(o[i], e[i + 1])
            res += [lo, hi]
        else:
            res.append(o[i])
    if len(e) > len(o) + 1:
        res += e[len(o) + 1:]
    return res


def _knn_block_kernel(q_ref, k_ref, radii_ref, idx_ref):
    q = q_ref[...]
    kt = k_ref[...]
    qq = jnp.sum(q * q, axis=1, keepdims=True)
    kk = jnp.sum(kt * kt, axis=1)[None, :]
    qk = lax.dot_general(q, kt, (((1,), (1,)), ((), ())),
                         preferred_element_type=jnp.float32)
    d2 = jnp.maximum(qq + kk - 2.0 * qk, 0.0)

    lane = lax.broadcasted_iota(jnp.int32, (BLOCK, LANES), 1)

    def build(lo, hi):
        if hi - lo == 1:
            v = d2[:, lo * LANES:(lo + 1) * LANES]
            return [_leaf(v, lane + lo * LANES)]
        mid = (lo + hi) // 2
        return _oe_merge(build(lo, mid), build(mid, hi))[:K + 1]

    elems = build(0, NCOL)
    lv = [el[0]() for el in elems]
    li = [el[1]() for el in elems]

    radii_acc = jnp.zeros((BLOCK, 1), dtype=jnp.float32)
    idx_cols = []
    for r in range(K + 1):
        mv = lv[0]
        for t in range(1, r + 1):
            mv = jnp.minimum(mv, lv[t])
        m = jnp.min(mv, axis=1, keepdims=True)
        ai = jnp.full((BLOCK, LANES), N, dtype=jnp.int32)
        for t in range(r + 1):
            ai = jnp.where(lv[t] == m, jnp.minimum(ai, li[t]), ai)
        am = jnp.min(ai, axis=1, keepdims=True)
        idx_cols.append(am)
        if r > 0:
            radii_acc = radii_acc + jnp.sqrt(jnp.maximum(m, 1e-12))
        if r < K:
            for t in range(r + 1):
                lv[t] = jnp.where(li[t] == am, jnp.float32(jnp.inf), lv[t])
    idx_ref[...] = jnp.concatenate(idx_cols, axis=1)
    radii_ref[...] = radii_acc * (1.0 / K)


def kernel(points, norms):
    pts = jnp.zeros((N, PAD_D), dtype=jnp.float32).at[:, :3].set(points)
    radii2d, idx = pl.pallas_call(
        _knn_block_kernel,
        grid=(N // BLOCK,),
        in_specs=[
            pl.BlockSpec((BLOCK, PAD_D), lambda i: (i, 0)),
            pl.BlockSpec((N, PAD_D), lambda i: (0, 0)),
        ],
        out_specs=[
            pl.BlockSpec((BLOCK, 1), lambda i: (i, 0)),
            pl.BlockSpec((BLOCK, K + 1), lambda i: (i, 0)),
        ],
        out_shape=[
            jax.ShapeDtypeStruct((N, 1), jnp.float32),
            jax.ShapeDtypeStruct((N, K + 1), jnp.int32),
        ],
    )(pts, pts)
    radii = radii2d[:, 0]
    src = jnp.repeat(jnp.arange(N, dtype=jnp.int32), K)
    dst = idx[:, 1:].reshape(-1)
    edge_index = jnp.stack([src, dst], axis=0)
    return points, norms, radii, edge_index


# speedup vs baseline: 1.5114x; 1.5114x over previous
import jax
import jax.numpy as jnp
from jax import lax
from jax.experimental import pallas as pl

N = 8192
K = 8
BLOCK = 512
PAD_D = 8
LANES = 128
NCOL = N // LANES
CAP = 6


def _lazy(f):
    memo = []

    def g():
        if not memo:
            memo.append(f())
        return memo[0]
    return g


def _leaf(v, i):
    return (lambda: v), (lambda: i)


def _cx(a, b):
    pred = _lazy(lambda: a[0]() <= b[0]())
    lo = (_lazy(lambda: jnp.minimum(a[0](), b[0]())),
          _lazy(lambda: jnp.where(pred(), a[1](), b[1]())))
    hi = (_lazy(lambda: jnp.maximum(a[0](), b[0]())),
          _lazy(lambda: jnp.where(pred(), b[1](), a[1]())))
    return lo, hi


def _oe_merge(a, b):
    if not a:
        return b
    if not b:
        return a
    if len(a) == 1 and len(b) == 1:
        lo, hi = _cx(a[0], b[0])
        return [lo, hi]
    e = _oe_merge(a[::2], b[::2])
    o = _oe_merge(a[1::2], b[1::2])
    res = [e[0]]
    for i in range(len(o)):
        if i + 1 < len(e):
            lo, hi = _cx(o[i], e[i + 1])
            res += [lo, hi]
        else:
            res.append(o[i])
    if len(e) > len(o) + 1:
        res += e[len(o) + 1:]
    return res


def _knn_block_kernel(q_ref, k_ref, radii_ref, idx_ref):
    q = q_ref[...]
    kt = k_ref[...]
    qq = jnp.sum(q * q, axis=1, keepdims=True)
    kk = jnp.sum(kt * kt, axis=1)[None, :]
    qk = lax.dot_general(q, kt, (((1,), (1,)), ((), ())),
                         preferred_element_type=jnp.float32)
    d2 = jnp.maximum(qq + kk - 2.0 * qk, 0.0)

    lane = lax.broadcasted_iota(jnp.int32, (BLOCK, LANES), 1)

    def build(lo, hi):
        if hi - lo == 1:
            v = d2[:, lo * LANES:(lo + 1) * LANES]
            return [_leaf(v, lane + lo * LANES)]
        mid = (lo + hi) // 2
        return _oe_merge(build(lo, mid), build(mid, hi))[:CAP]

    elems = build(0, NCOL)
    lv = [el[0]() for el in elems]
    li = [el[1]() for el in elems]

    radii_acc = jnp.zeros((BLOCK, 1), dtype=jnp.float32)
    idx_cols = []
    for r in range(K + 1):
        w = min(r + 1, CAP)
        mv = lv[0]
        for t in range(1, w):
            mv = jnp.minimum(mv, lv[t])
        m = jnp.min(mv, axis=1, keepdims=True)
        ai = jnp.full((BLOCK, LANES), N, dtype=jnp.int32)
        for t in range(w):
            ai = jnp.where(lv[t] == m, jnp.minimum(ai, li[t]), ai)
        am = jnp.min(ai, axis=1, keepdims=True)
        idx_cols.append(am)
        if r > 0:
            radii_acc = radii_acc + jnp.sqrt(jnp.maximum(m, 1e-12))
        if r < K:
            for t in range(w):
                lv[t] = jnp.where(li[t] == am, jnp.float32(jnp.inf), lv[t])
    idx_ref[...] = jnp.concatenate(idx_cols, axis=1)
    radii_ref[...] = radii_acc * (1.0 / K)


def kernel(points, norms):
    pts = jnp.zeros((N, PAD_D), dtype=jnp.float32).at[:, :3].set(points)
    radii2d, idx = pl.pallas_call(
        _knn_block_kernel,
        grid=(N // BLOCK,),
        in_specs=[
            pl.BlockSpec((BLOCK, PAD_D), lambda i: (i, 0)),
            pl.BlockSpec((N, PAD_D), lambda i: (0, 0)),
        ],
        out_specs=[
            pl.BlockSpec((BLOCK, 1), lambda i: (i, 0)),
            pl.BlockSpec((BLOCK, K + 1), lambda i: (i, 0)),
        ],
        out_shape=[
            jax.ShapeDtypeStruct((N, 1), jnp.float32),
            jax.ShapeDtypeStruct((N, K + 1), jnp.int32),
        ],
    )(pts, pts)
    radii = radii2d[:, 0]
    src = jnp.repeat(jnp.arange(N, dtype=jnp.int32), K)
    dst = idx[:, 1:].reshape(-1)
    edge_index = jnp.stack([src, dst], axis=0)
    return points, norms, radii, edge_index


# speedup vs baseline: 1.7199x; 1.1380x over previous
import jax
import jax.numpy as jnp
from jax import lax
from jax.experimental import pallas as pl

N = 8192
K = 8
BLOCK = 512
PAD_D = 8
LANES = 128
NCOL = N // LANES
CAP = 5


def _lazy(f):
    memo = []

    def g():
        if not memo:
            memo.append(f())
        return memo[0]
    return g


def _leaf(v, i):
    return (lambda: v), (lambda: i)


def _cx(a, b):
    pred = _lazy(lambda: a[0]() <= b[0]())
    lo = (_lazy(lambda: jnp.minimum(a[0](), b[0]())),
          _lazy(lambda: jnp.where(pred(), a[1](), b[1]())))
    hi = (_lazy(lambda: jnp.maximum(a[0](), b[0]())),
          _lazy(lambda: jnp.where(pred(), b[1](), a[1]())))
    return lo, hi


def _oe_merge(a, b):
    if not a:
        return b
    if not b:
        return a
    if len(a) == 1 and len(b) == 1:
        lo, hi = _cx(a[0], b[0])
        return [lo, hi]
    e = _oe_merge(a[::2], b[::2])
    o = _oe_merge(a[1::2], b[1::2])
    res = [e[0]]
    for i in range(len(o)):
        if i + 1 < len(e):
            lo, hi = _cx(o[i], e[i + 1])
            res += [lo, hi]
        else:
            res.append(o[i])
    if len(e) > len(o) + 1:
        res += e[len(o) + 1:]
    return res


def _knn_block_kernel(q_ref, k_ref, radii_ref, idx_ref):
    q = q_ref[...]
    kt = k_ref[...]
    qq = jnp.sum(q * q, axis=1, keepdims=True)
    kk = jnp.sum(kt * kt, axis=1)[None, :]
    qk = lax.dot_general(q, kt, (((1,), (1,)), ((), ())),
                         preferred_element_type=jnp.float32)
    d2 = jnp.maximum(qq + kk - 2.0 * qk, 0.0)

    lane = lax.broadcasted_iota(jnp.int32, (BLOCK, LANES), 1)

    def build(lo, hi):
        if hi - lo == 1:
            v = d2[:, lo * LANES:(lo + 1) * LANES]
            return [_leaf(v, lane + lo * LANES)]
        mid = (lo + hi) // 2
        return _oe_merge(build(lo, mid), build(mid, hi))[:CAP]

    elems = build(0, NCOL)
    lv = [el[0]() for el in elems]
    li = [el[1]() for el in elems]

    radii_acc = jnp.zeros((BLOCK, 1), dtype=jnp.float32)
    idx_cols = []
    for r in range(K + 1):
        w = min(r + 1, CAP)
        mv = lv[0]
        for t in range(1, w):
            mv = jnp.minimum(mv, lv[t])
        m = jnp.min(mv, axis=1, keepdims=True)
        ai = jnp.full((BLOCK, LANES), N, dtype=jnp.int32)
        for t in range(w):
            ai = jnp.where(lv[t] == m, jnp.minimum(ai, li[t]), ai)
        am = jnp.min(ai, axis=1, keepdims=True)
        idx_cols.append(am)
        if r > 0:
            radii_acc = radii_acc + jnp.sqrt(jnp.maximum(m, 1e-12))
        if r < K:
            for t in range(w):
                lv[t] = jnp.where(li[t] == am, jnp.float32(jnp.inf), lv[t])
    idx_ref[...] = jnp.concatenate(idx_cols, axis=1)
    radii_ref[...] = radii_acc * (1.0 / K)


def kernel(points, norms):
    pts = jnp.zeros((N, PAD_D), dtype=jnp.float32).at[:, :3].set(points)
    radii2d, idx = pl.pallas_call(
        _knn_block_kernel,
        grid=(N // BLOCK,),
        in_specs=[
            pl.BlockSpec((BLOCK, PAD_D), lambda i: (i, 0)),
            pl.BlockSpec((N, PAD_D), lambda i: (0, 0)),
        ],
        out_specs=[
            pl.BlockSpec((BLOCK, 1), lambda i: (i, 0)),
            pl.BlockSpec((BLOCK, K + 1), lambda i: (i, 0)),
        ],
        out_shape=[
            jax.ShapeDtypeStruct((N, 1), jnp.float32),
            jax.ShapeDtypeStruct((N, K + 1), jnp.int32),
        ],
    )(pts, pts)
    radii = radii2d[:, 0]
    src = jnp.repeat(jnp.arange(N, dtype=jnp.int32), K)
    dst = idx[:, 1:].reshape(-1)
    edge_index = jnp.stack([src, dst], axis=0)
    return points, norms, radii, edge_index


# speedup vs baseline: 1.8309x; 1.0645x over previous
import jax
import jax.numpy as jnp
from jax import lax
from jax.experimental import pallas as pl

N = 8192
K = 8
BLOCK = 512
PAD_D = 8
LANES = 128
NCOL = N // LANES
CAP = 4


def _lazy(f):
    memo = []

    def g():
        if not memo:
            memo.append(f())
        return memo[0]
    return g


def _leaf(v, i):
    return (lambda: v), (lambda: i)


def _cx(a, b):
    pred = _lazy(lambda: a[0]() <= b[0]())
    lo = (_lazy(lambda: jnp.minimum(a[0](), b[0]())),
          _lazy(lambda: jnp.where(pred(), a[1](), b[1]())))
    hi = (_lazy(lambda: jnp.maximum(a[0](), b[0]())),
          _lazy(lambda: jnp.where(pred(), b[1](), a[1]())))
    return lo, hi


def _oe_merge(a, b):
    if not a:
        return b
    if not b:
        return a
    if len(a) == 1 and len(b) == 1:
        lo, hi = _cx(a[0], b[0])
        return [lo, hi]
    e = _oe_merge(a[::2], b[::2])
    o = _oe_merge(a[1::2], b[1::2])
    res = [e[0]]
    for i in range(len(o)):
        if i + 1 < len(e):
            lo, hi = _cx(o[i], e[i + 1])
            res += [lo, hi]
        else:
            res.append(o[i])
    if len(e) > len(o) + 1:
        res += e[len(o) + 1:]
    return res


def _knn_block_kernel(q_ref, k_ref, radii_ref, idx_ref):
    q = q_ref[...]
    kt = k_ref[...]
    qq = jnp.sum(q * q, axis=1, keepdims=True)
    kk = jnp.sum(kt * kt, axis=1)[None, :]
    qk = lax.dot_general(q, kt, (((1,), (1,)), ((), ())),
                         preferred_element_type=jnp.float32)
    d2 = jnp.maximum(qq + kk - 2.0 * qk, 0.0)

    lane = lax.broadcasted_iota(jnp.int32, (BLOCK, LANES), 1)

    def build(lo, hi):
        if hi - lo == 1:
            v = d2[:, lo * LANES:(lo + 1) * LANES]
            return [_leaf(v, lane + lo * LANES)]
        mid = (lo + hi) // 2
        return _oe_merge(build(lo, mid), build(mid, hi))[:CAP]

    elems = build(0, NCOL)
    lv = [el[0]() for el in elems]
    li = [el[1]() for el in elems]

    radii_acc = jnp.zeros((BLOCK, 1), dtype=jnp.float32)
    idx_cols = []
    for r in range(K + 1):
        w = min(r + 1, CAP)
        mv = lv[0]
        for t in range(1, w):
            mv = jnp.minimum(mv, lv[t])
        m = jnp.min(mv, axis=1, keepdims=True)
        ai = jnp.full((BLOCK, LANES), N, dtype=jnp.int32)
        for t in range(w):
            ai = jnp.where(lv[t] == m, jnp.minimum(ai, li[t]), ai)
        am = jnp.min(ai, axis=1, keepdims=True)
        idx_cols.append(am)
        if r > 0:
            radii_acc = radii_acc + jnp.sqrt(jnp.maximum(m, 1e-12))
        if r < K:
            for t in range(w):
                lv[t] = jnp.where(li[t] == am, jnp.float32(jnp.inf), lv[t])
    idx_ref[...] = jnp.concatenate(idx_cols, axis=1)
    radii_ref[...] = radii_acc * (1.0 / K)


def kernel(points, norms):
    pts = jnp.zeros((N, PAD_D), dtype=jnp.float32).at[:, :3].set(points)
    radii2d, idx = pl.pallas_call(
        _knn_block_kernel,
        grid=(N // BLOCK,),
        in_specs=[
            pl.BlockSpec((BLOCK, PAD_D), lambda i: (i, 0)),
            pl.BlockSpec((N, PAD_D), lambda i: (0, 0)),
        ],
        out_specs=[
            pl.BlockSpec((BLOCK, 1), lambda i: (i, 0)),
            pl.BlockSpec((BLOCK, K + 1), lambda i: (i, 0)),
        ],
        out_shape=[
            jax.ShapeDtypeStruct((N, 1), jnp.float32),
            jax.ShapeDtypeStruct((N, K + 1), jnp.int32),
        ],
    )(pts, pts)
    radii = radii2d[:, 0]
    src = jnp.repeat(jnp.arange(N, dtype=jnp.int32), K)
    dst = idx[:, 1:].reshape(-1)
    edge_index = jnp.stack([src, dst], axis=0)
    return points, norms, radii, edge_index
